# baseline (speedup 1.0000x reference)
import jax
import jax.numpy as jnp
from jax import lax
from jax.experimental import pallas as pl
from jax.experimental.pallas import tpu as pltpu
from jax.experimental.pallas import tpu_sc as plsc

_NC = 2
_NS = 16
_NW = _NC * _NS

_CH = 80
_NB = 10
_NSP = 2
_K = 7


def _gather_kernel(table, idxh, out, idx_v, bufs, spbufs, gsem, wsem):
    b_per_w = idxh.shape[0] // _NW
    n_chunks = b_per_w // _CH
    n_groups = n_chunks // _NB
    sid = lax.axis_index("s")
    wid = sid * _NC + lax.axis_index("c")
    base = wid * b_per_w
    spb = spbufs.at[sid]
    pltpu.sync_copy(idxh.at[pl.ds(base, b_per_w)], idx_v)

    def idx_slice(c):
        return idx_v.at[pl.ds(c * _CH, _CH)]

    def fire_gather(c, b):
        pltpu.async_copy(table.at[idx_slice(c)], bufs.at[b], gsem.at[b])

    def wait_gather(c, b):
        pltpu.make_async_copy(table.at[idx_slice(c)], bufs.at[b],
                              gsem.at[b]).wait()

    def fire_wb(c, b):
        s = b % _NSP
        pltpu.sync_copy(bufs.at[b], spb.at[s])
        pltpu.async_copy(spb.at[s], out.at[pl.ds(base + c * _CH, _CH)],
                         wsem.at[s])

    def wait_wb(c, b):
        s = b % _NSP
        pltpu.make_async_copy(spb.at[s], out.at[pl.ds(base + c * _CH, _CH)],
                              wsem.at[s]).wait()

    for c in range(_K):
        fire_gather(c, c % _NB)

    for b in range(_NB):
        i = b
        bb = (b + _K) % _NB
        fire_gather(i + _K, bb)
        wait_gather(i, b)
        if i >= _NSP:
            wait_wb(i - _NSP, b)
        fire_wb(i, b)

    def group_body(g, carry):
        for b in range(_NB):
            i = g * _NB + b
            bb = (b + _K) % _NB
            fire_gather(i + _K, bb)
            wait_gather(i, b)
            wait_wb(i - _NSP, b)
            fire_wb(i, b)
        return carry

    lax.fori_loop(1, n_groups - 1, group_body, 0, unroll=False)

    for b in range(_NB):
        i = (n_groups - 1) * _NB + b
        if i + _K < n_chunks:
            bb = (b + _K) % _NB
            fire_gather(i + _K, bb)
        wait_gather(i, b)
        wait_wb(i - _NSP, b)
        fire_wb(i, b)

    for b in range(_NSP):
        wait_wb(n_chunks - _NSP + b, _NB - _NSP + b)


@jax.jit
def _embedding_lookup(weight, flat_ids):
    b_total = flat_ids.shape[0]
    d = weight.shape[1]
    b_per_w = b_total // _NW
    mesh = plsc.VectorSubcoreMesh(core_axis_name="c", subcore_axis_name="s")
    f = pl.kernel(
        _gather_kernel,
        out_type=jax.ShapeDtypeStruct((b_total, d), jnp.float32),
        mesh=mesh,
        scratch_types=[
            pltpu.VMEM((b_per_w,), jnp.int32),
            pltpu.VMEM((_NB, _CH, d), jnp.float32),
            pltpu.VMEM_SHARED((_NS, _NSP, _CH, d), jnp.float32),
            pltpu.SemaphoreType.DMA((_NB,)),
            pltpu.SemaphoreType.DMA((_NSP,)),
        ],
    )
    return f(weight, flat_ids)


def kernel(token_ids, weight):
    b, l = token_ids.shape
    flat = token_ids.reshape(-1).astype(jnp.int32)
    out = _embedding_lookup(weight, flat)
    return out.reshape(b, l, weight.shape[1])
